# Initial kernel scaffold; baseline (speedup 1.0000x reference)
import jax
import jax.numpy as jnp
from jax.experimental import pallas as pl


def kernel(input, table):
    raise NotImplementedError("write your pallas kernel here")


# speedup vs baseline: 2.7592x; 2.7592x over previous
import functools

import jax
import jax.numpy as jnp
from jax import lax
from jax.experimental import pallas as pl
from jax.experimental.pallas import tpu as pltpu
from jax.experimental.pallas import tpu_sc as plsc

B = 16384
SEQ = 50
D = 32
NW = 32
RPW = B // NW
CB = 32
NCH = RPW // CB
IPC = CB * SEQ
G = 80
NG = IPC // G

_mesh = plsc.VectorSubcoreMesh(core_axis_name="c", subcore_axis_name="s")


@functools.partial(
    pl.kernel,
    mesh=_mesh,
    out_type=jax.ShapeDtypeStruct((B, D), jnp.float32),
    compiler_params=pltpu.CompilerParams(use_tc_tiling_on_sc=False),
    scratch_types=[
        pltpu.VMEM((IPC,), jnp.int32),
        pltpu.VMEM((IPC, D), jnp.float32),
        pltpu.VMEM((CB, D), jnp.float32),
        pltpu.SemaphoreType.DMA,
    ],
)
def _sum_embed(idx_hbm, table_hbm, out_hbm, idx_v, rows_v, out_v, sem):
    ci = lax.axis_index("c")
    si = lax.axis_index("s")
    wid = si * 2 + ci

    def chunk_body(c, carry):
        base_row = wid * RPW + c * CB
        pltpu.sync_copy(idx_hbm.at[pl.ds(base_row * SEQ, IPC)], idx_v)
        cps = [
            pltpu.async_copy(
                table_hbm.at[idx_v.at[pl.ds(g * G, G)]],
                rows_v.at[pl.ds(g * G, G)],
                sem,
            )
            for g in range(NG)
        ]
        for cp in cps:
            cp.wait()

        def row_body(r, carry2):
            rb = r * SEQ
            a0 = rows_v[rb, pl.ds(0, 16)]
            a1 = rows_v[rb, pl.ds(16, 16)]
            b0 = rows_v[rb + 1, pl.ds(0, 16)]
            b1 = rows_v[rb + 1, pl.ds(16, 16)]
            for j in range(2, SEQ, 2):
                a0 = a0 + rows_v[rb + j, pl.ds(0, 16)]
                a1 = a1 + rows_v[rb + j, pl.ds(16, 16)]
                b0 = b0 + rows_v[rb + j + 1, pl.ds(0, 16)]
                b1 = b1 + rows_v[rb + j + 1, pl.ds(16, 16)]
            out_v[r, pl.ds(0, 16)] = a0 + b0
            out_v[r, pl.ds(16, 16)] = a1 + b1
            return carry2

        lax.fori_loop(0, CB, row_body, 0)
        pltpu.sync_copy(out_v, out_hbm.at[pl.ds(base_row, CB)])
        return carry

    lax.fori_loop(0, NCH, chunk_body, 0)


def kernel(input, table):
    idx = input.astype(jnp.int32).reshape(-1)
    return _sum_embed(idx, table)


# speedup vs baseline: 2.9195x; 1.0581x over previous
import functools

import jax
import jax.numpy as jnp
from jax import lax
from jax.experimental import pallas as pl
from jax.experimental.pallas import tpu as pltpu
from jax.experimental.pallas import tpu_sc as plsc

B = 16384
SEQ = 50
D = 32
NW = 32
RPW = B // NW
CB = 32
NCH = RPW // CB

_mesh = plsc.VectorSubcoreMesh(core_axis_name="c", subcore_axis_name="s")


@functools.partial(
    pl.kernel,
    mesh=_mesh,
    out_type=jax.ShapeDtypeStruct((B, D), jnp.float32),
    scratch_types=[
        pltpu.VMEM((2, CB, SEQ), jnp.int32),
        pltpu.VMEM((CB, SEQ, D), jnp.float32),
        pltpu.VMEM((CB, SEQ, D), jnp.float32),
        pltpu.VMEM((CB, D), jnp.float32),
        pltpu.SemaphoreType.DMA,
        pltpu.SemaphoreType.DMA,
    ],
    compiler_params=pltpu.CompilerParams(use_tc_tiling_on_sc=False),
)
def _sum_embed(idx_hbm, t_hbm, out_hbm, gidx_v, rows0_v, rows1_v, out_v,
               sem0, sem1):
    ci = lax.axis_index("c")
    si = lax.axis_index("s")
    wid = si * 2 + ci
    rbase = wid * RPW

    rows_bufs = (rows0_v, rows1_v)
    sems = (sem0, sem1)

    def fire(c, par):
        pltpu.sync_copy(idx_hbm.at[pl.ds(rbase + c * CB, CB)],
                        gidx_v.at[par])

        def g_body(r, carry):
            pltpu.async_copy(
                t_hbm.at[gidx_v.at[par, r]],
                rows_bufs[par].at[r],
                sems[par],
            )
            return carry

        lax.fori_loop(0, CB, g_body, 0)

    def drain(par):
        def w_body(r, carry):
            pltpu.make_async_copy(
                t_hbm.at[pl.ds(0, SEQ)],
                rows_bufs[par].at[r],
                sems[par],
            ).wait()
            return carry

        lax.fori_loop(0, CB, w_body, 0)

    def accumulate(c, par):
        rows_v = rows_bufs[par]

        def row_body(r, carry):
            a0 = rows_v[r, 0, pl.ds(0, 16)]
            a1 = rows_v[r, 0, pl.ds(16, 16)]
            b0 = rows_v[r, 1, pl.ds(0, 16)]
            b1 = rows_v[r, 1, pl.ds(16, 16)]
            for j in range(2, SEQ, 2):
                a0 = a0 + rows_v[r, j, pl.ds(0, 16)]
                a1 = a1 + rows_v[r, j, pl.ds(16, 16)]
                b0 = b0 + rows_v[r, j + 1, pl.ds(0, 16)]
                b1 = b1 + rows_v[r, j + 1, pl.ds(16, 16)]
            out_v[r, pl.ds(0, 16)] = a0 + b0
            out_v[r, pl.ds(16, 16)] = a1 + b1
            return carry

        lax.fori_loop(0, CB, row_body, 0)
        pltpu.sync_copy(out_v, out_hbm.at[pl.ds(rbase + c * CB, CB)])

    fire(0, 0)

    def pair_body(p, carry):
        ca = 2 * p
        fire(ca + 1, 1)
        drain(0)
        accumulate(ca, 0)

        @pl.when(p < NCH // 2 - 1)
        def _():
            fire(ca + 2, 0)

        drain(1)
        accumulate(ca + 1, 1)
        return carry

    lax.fori_loop(0, NCH // 2, pair_body, 0)


def kernel(input, table):
    idx = input.astype(jnp.int32)
    return _sum_embed(idx, table)


# speedup vs baseline: 2.9255x; 1.0021x over previous
import functools

import jax
import jax.numpy as jnp
from jax import lax
from jax.experimental import pallas as pl
from jax.experimental.pallas import tpu as pltpu
from jax.experimental.pallas import tpu_sc as plsc

B = 16384
SEQ = 50
D = 32
NW = 32
RPW = B // NW
CB = 32
NCH = RPW // CB
G = 100
SPC = CB * SEQ // G

_mesh = plsc.VectorSubcoreMesh(core_axis_name="c", subcore_axis_name="s")


@functools.partial(
    pl.kernel,
    mesh=_mesh,
    out_type=jax.ShapeDtypeStruct((B, D), jnp.float32),
    scratch_types=[
        pltpu.VMEM((2, SPC, G), jnp.int32),
        pltpu.VMEM((SPC, G, D), jnp.float32),
        pltpu.VMEM((SPC, G, D), jnp.float32),
        pltpu.VMEM((CB, D), jnp.float32),
        pltpu.SemaphoreType.DMA,
        pltpu.SemaphoreType.DMA,
    ],
    compiler_params=pltpu.CompilerParams(use_tc_tiling_on_sc=False),
)
def _sum_embed(idx_hbm, t_hbm, out_hbm, gidx_v, rows0_v, rows1_v, out_v,
               sem0, sem1):
    ci = lax.axis_index("c")
    si = lax.axis_index("s")
    wid = si * 2 + ci
    rbase = wid * RPW

    rows_bufs = (rows0_v, rows1_v)
    sems = (sem0, sem1)

    def fire(c, par):
        pltpu.sync_copy(idx_hbm.at[pl.ds((rbase + c * CB) * SEQ // G, SPC)],
                        gidx_v.at[par])

        def g_body(g, carry):
            pltpu.async_copy(
                t_hbm.at[gidx_v.at[par, g]],
                rows_bufs[par].at[g],
                sems[par],
            )
            return carry

        lax.fori_loop(0, SPC, g_body, 0)

    def drain(par):
        def w_body(g, carry):
            pltpu.make_async_copy(
                t_hbm.at[pl.ds(0, G)],
                rows_bufs[par].at[g],
                sems[par],
            ).wait()
            return carry

        lax.fori_loop(0, SPC, w_body, 0)

    def accumulate(c, par):
        rows_v = rows_bufs[par]

        def row_body(r, carry):
            g = r // 2
            p = (r % 2) * SEQ
            a0 = rows_v[g, p + 0, pl.ds(0, 16)]
            a1 = rows_v[g, p + 0, pl.ds(16, 16)]
            b0 = rows_v[g, p + 1, pl.ds(0, 16)]
            b1 = rows_v[g, p + 1, pl.ds(16, 16)]
            for j in range(2, SEQ, 2):
                a0 = a0 + rows_v[g, p + j, pl.ds(0, 16)]
                a1 = a1 + rows_v[g, p + j, pl.ds(16, 16)]
                b0 = b0 + rows_v[g, p + j + 1, pl.ds(0, 16)]
                b1 = b1 + rows_v[g, p + j + 1, pl.ds(16, 16)]
            out_v[r, pl.ds(0, 16)] = a0 + b0
            out_v[r, pl.ds(16, 16)] = a1 + b1
            return carry

        lax.fori_loop(0, CB, row_body, 0)
        pltpu.sync_copy(out_v, out_hbm.at[pl.ds(rbase + c * CB, CB)])

    fire(0, 0)

    def pair_body(p, carry):
        ca = 2 * p
        fire(ca + 1, 1)
        drain(0)
        accumulate(ca, 0)

        @pl.when(p < NCH // 2 - 1)
        def _():
            fire(ca + 2, 0)

        drain(1)
        accumulate(ca + 1, 1)
        return carry

    lax.fori_loop(0, NCH // 2, pair_body, 0)


def kernel(input, table):
    idx = input.astype(jnp.int32).reshape(B * SEQ // G, G)
    return _sum_embed(idx, table)
